# baseline (speedup 1.0000x reference)
import functools

import jax
import jax.numpy as jnp
from jax.experimental import pallas as pl
from jax.experimental.pallas import tpu as pltpu


def _gcn_block_kernel(adj_ref, node_ref, w_ref, b_ref, out_ref):
    tmp = jnp.dot(adj_ref[...], node_ref[...],
                  preferred_element_type=jnp.float32)
    out = jnp.dot(tmp, w_ref[...], preferred_element_type=jnp.float32)
    out_ref[...] = jnp.maximum(out, 0.0) + b_ref[...]


@functools.partial(jax.jit, static_argnames=())
def kernel(node, adj, weight, bias):
    m, n = adj.shape
    _, d_in = node.shape
    d_out = weight.shape[1]
    bias2d = bias.reshape(1, d_out)

    bm = 280
    grid = (pl.cdiv(m, bm),)

    return pl.pallas_call(
        _gcn_block_kernel,
        grid=grid,
        in_specs=[
            pl.BlockSpec((bm, n), lambda i: (i, 0)),
            pl.BlockSpec((n, d_in), lambda i: (0, 0)),
            pl.BlockSpec((d_in, d_out), lambda i: (0, 0)),
            pl.BlockSpec((1, d_out), lambda i: (0, 0)),
        ],
        out_specs=pl.BlockSpec((bm, d_out), lambda i: (i, 0)),
        out_shape=jax.ShapeDtypeStruct((m, d_out), jnp.float32),
        compiler_params=pltpu.CompilerParams(
            dimension_semantics=("parallel",),
        ),
    )(adj, node, weight, bias2d)


# speedup vs baseline: 1.0041x; 1.0041x over previous
import functools

import jax
import jax.numpy as jnp
from jax.experimental import pallas as pl
from jax.experimental.pallas import tpu as pltpu


def _gcn_block_kernel(adj_ref, node_ref, w_ref, b_ref, out_ref):
    tmp = jnp.dot(adj_ref[...], node_ref[...],
                  preferred_element_type=jnp.float32)
    out = jnp.dot(tmp, w_ref[...], preferred_element_type=jnp.float32)
    out_ref[...] = jnp.maximum(out, 0.0) + b_ref[...]


@functools.partial(jax.jit, static_argnames=())
def kernel(node, adj, weight, bias):
    m, n = adj.shape
    _, d_in = node.shape
    d_out = weight.shape[1]
    bias2d = bias.reshape(1, d_out)

    bm = 264
    grid = (pl.cdiv(m, bm),)

    return pl.pallas_call(
        _gcn_block_kernel,
        grid=grid,
        in_specs=[
            pl.BlockSpec((bm, n), lambda i: (i, 0)),
            pl.BlockSpec((n, d_in), lambda i: (0, 0)),
            pl.BlockSpec((d_in, d_out), lambda i: (0, 0)),
            pl.BlockSpec((1, d_out), lambda i: (0, 0)),
        ],
        out_specs=pl.BlockSpec((bm, d_out), lambda i: (i, 0)),
        out_shape=jax.ShapeDtypeStruct((m, d_out), jnp.float32),
        compiler_params=pltpu.CompilerParams(
            dimension_semantics=("parallel",),
        ),
    )(adj, node, weight, bias2d)
